# baseline (speedup 1.0000x reference)
import functools

import jax
import jax.numpy as jnp
from jax import lax
from jax.experimental import pallas as pl
from jax.experimental.pallas import tpu as pltpu
from jax.experimental.pallas import tpu_sc as plsc

N = 16384
M = 4096
CF = 128
CC = 256
K = 3
FW = 384
BLK = 256
GRID = N // BLK
CW = 128
NCH = M // CW


def _knn_body(q_ref, s_ref, idx_ref):
    q = q_ref[...]
    s = s_ref[...]
    qq = (q[:, 0:1] * q[:, 0:1] + q[:, 1:2] * q[:, 1:2]) + q[:, 2:3] * q[:, 2:3]
    ss = (s[0:1] * s[0:1] + s[1:2] * s[1:2]) + s[2:3] * s[2:3]
    qb = q.astype(jnp.bfloat16)
    sb = s.astype(jnp.bfloat16)
    qs = jnp.dot(qb, sb, preferred_element_type=jnp.float32)
    inf = jnp.float32(float("inf"))
    big = jnp.int32(2**30)
    v1 = jnp.full((BLK, CW), inf, jnp.float32)
    v2 = v1
    v3 = v1
    i1 = jnp.full((BLK, CW), big, jnp.int32)
    i2 = i1
    i3 = i1
    lane = lax.broadcasted_iota(jnp.int32, (BLK, CW), 1)
    for c in range(NCH):
        sl = slice(c * CW, (c + 1) * CW)
        v = (qq + ss[:, sl]) - 2.0 * qs[:, sl]
        iv = lane + (c * CW)
        lt1 = v < v1
        lt2 = v < v2
        lt3 = v < v3
        v3n = jnp.where(lt3, jnp.where(lt2, v2, v), v3)
        i3n = jnp.where(lt3, jnp.where(lt2, i2, iv), i3)
        v2n = jnp.where(lt2, jnp.where(lt1, v1, v), v2)
        i2n = jnp.where(lt2, jnp.where(lt1, i1, iv), i2)
        v1 = jnp.where(lt1, v, v1)
        i1 = jnp.where(lt1, iv, i1)
        v2, v3, i2, i3 = v2n, v3n, i2n, i3n
    V = jnp.concatenate([v1, v2, v3], axis=1)
    I = jnp.concatenate([i1, i2, i3], axis=1)
    idxs = []
    for _ in range(K):
        m = jnp.min(V, axis=1, keepdims=True)
        a = jnp.min(jnp.where(V == m, I, big), axis=1, keepdims=True)
        idxs.append(a)
        V = jnp.where((V == m) & (I == a), inf, V)
    idx_ref[...] = jnp.concatenate(idxs, axis=1)


def _knn(q, s):
    return pl.pallas_call(
        _knn_body,
        grid=(GRID,),
        in_specs=[
            pl.BlockSpec((BLK, 3), lambda i: (i, 0)),
            pl.BlockSpec((3, M), lambda i: (0, 0)),
        ],
        out_specs=pl.BlockSpec((BLK, K), lambda i: (i, 0)),
        out_shape=jax.ShapeDtypeStruct((N, K), jnp.int32),
    )(q, s)


_NC = 2
_NS = 16
_NW = _NC * _NS
_ROWS = K * N
_RPW = _ROWS // _NW
_CHUNK = 128
_NCHUNK = _RPW // _CHUNK


def _sc_gather(table, idx_flat):
    mesh = plsc.VectorSubcoreMesh(core_axis_name="c", subcore_axis_name="s")

    @functools.partial(
        pl.kernel,
        mesh=mesh,
        out_type=jax.ShapeDtypeStruct((_ROWS, FW), jnp.float32),
        scratch_types=[
            pltpu.VMEM((_RPW,), jnp.int32),
            pltpu.VMEM((_CHUNK, FW), jnp.float32),
            pltpu.VMEM((_CHUNK, FW), jnp.float32),
            pltpu.SemaphoreType.DMA,
            pltpu.SemaphoreType.DMA,
        ],
    )
    def k(table_hbm, idx_hbm, out_hbm, idx_v, b0, b1, s0, s1):
        wid = lax.axis_index("s") * _NC + lax.axis_index("c")
        base = wid * _RPW
        pltpu.sync_copy(idx_hbm.at[pl.ds(base, _RPW)], idx_v)
        bufs = (b0, b1)
        sems = (s0, s1)

        def issue(t, b):
            isl = idx_v.at[pl.ds(t * _CHUNK, _CHUNK)]
            return pltpu.async_copy(table_hbm.at[isl], bufs[b], sems[b])

        cps = [None, None]
        cps[0] = issue(0, 0)
        for t in range(_NCHUNK):
            b = t % 2
            nb = (t + 1) % 2
            if t + 1 < _NCHUNK:
                cps[nb] = issue(t + 1, nb)
            cps[b].wait()
            pltpu.sync_copy(bufs[b], out_hbm.at[pl.ds(base + t * _CHUNK, _CHUNK)])

    return k(table, idx_flat)


def _mlp1_body(g_ref, qp_ref, ff_ref, w1a_ref, w1b_ref,
               y1_ref, s_ref, q_ref):
    i = pl.program_id(0)
    qp = qp_ref[...]
    ws = []
    for j in range(K):
        t0 = qp[:, 0:1] - g_ref[:, j, CC : CC + 1]
        t1 = qp[:, 1:2] - g_ref[:, j, CC + 1 : CC + 2]
        t2 = qp[:, 2:3] - g_ref[:, j, CC + 2 : CC + 3]
        d2 = (t0 * t0 + t1 * t1) + t2 * t2
        d = jnp.maximum(jnp.sqrt(d2), 1e-8)
        ws.append(1.0 / d)
    tot = (ws[0] + ws[1]) + ws[2]
    wn = [wj / tot for wj in ws]
    interp = (wn[0] * g_ref[:, 0, :CC] + wn[1] * g_ref[:, 1, :CC]) \
        + wn[2] * g_ref[:, 2, :CC]
    y = jnp.dot(interp, w1a_ref[...], preferred_element_type=jnp.float32)
    y = y + lax.dot_general(
        ff_ref[...], w1b_ref[...],
        dimension_numbers=(((0,), (0,)), ((), ())),
        preferred_element_type=jnp.float32)
    y1_ref[...] = y

    @pl.when(i == 0)
    def _init():
        s_ref[...] = jnp.zeros_like(s_ref)
        q_ref[...] = jnp.zeros_like(q_ref)

    s_ref[...] += jnp.sum(y, axis=0, keepdims=True)
    q_ref[...] += jnp.sum(y * y, axis=0, keepdims=True)


def _mlp1(g, q, ff, w1a_t, w1b_t):
    return pl.pallas_call(
        _mlp1_body,
        grid=(GRID,),
        in_specs=[
            pl.BlockSpec((BLK, K, FW), lambda i: (i, 0, 0)),
            pl.BlockSpec((BLK, 3), lambda i: (i, 0)),
            pl.BlockSpec((CF, BLK), lambda i: (0, i)),
            pl.BlockSpec((CC, 256), lambda i: (0, 0)),
            pl.BlockSpec((CF, 256), lambda i: (0, 0)),
        ],
        out_specs=[
            pl.BlockSpec((BLK, 256), lambda i: (i, 0)),
            pl.BlockSpec((1, 256), lambda i: (0, 0)),
            pl.BlockSpec((1, 256), lambda i: (0, 0)),
        ],
        out_shape=[
            jax.ShapeDtypeStruct((N, 256), jnp.float32),
            jax.ShapeDtypeStruct((1, 256), jnp.float32),
            jax.ShapeDtypeStruct((1, 256), jnp.float32),
        ],
    )(g, q, ff, w1a_t, w1b_t)


def _mlp2_body(y1_ref, a_ref, b_ref, w2_ref, y2_ref, s_ref, q_ref):
    i = pl.program_id(0)
    h = jnp.maximum(y1_ref[...] * a_ref[...] + b_ref[...], 0.0)
    y = jnp.dot(h, w2_ref[...], preferred_element_type=jnp.float32)
    y2_ref[...] = y

    @pl.when(i == 0)
    def _init():
        s_ref[...] = jnp.zeros_like(s_ref)
        q_ref[...] = jnp.zeros_like(q_ref)

    s_ref[...] += jnp.sum(y, axis=0, keepdims=True)
    q_ref[...] += jnp.sum(y * y, axis=0, keepdims=True)


def _mlp2(y1, a1, b1, w2_t):
    return pl.pallas_call(
        _mlp2_body,
        grid=(GRID,),
        in_specs=[
            pl.BlockSpec((BLK, 256), lambda i: (i, 0)),
            pl.BlockSpec((1, 256), lambda i: (0, 0)),
            pl.BlockSpec((1, 256), lambda i: (0, 0)),
            pl.BlockSpec((256, 256), lambda i: (0, 0)),
        ],
        out_specs=[
            pl.BlockSpec((BLK, 256), lambda i: (i, 0)),
            pl.BlockSpec((1, 256), lambda i: (0, 0)),
            pl.BlockSpec((1, 256), lambda i: (0, 0)),
        ],
        out_shape=[
            jax.ShapeDtypeStruct((N, 256), jnp.float32),
            jax.ShapeDtypeStruct((1, 256), jnp.float32),
            jax.ShapeDtypeStruct((1, 256), jnp.float32),
        ],
    )(y1, a1, b1, w2_t)


def _norm_body(y2_ref, a_ref, b_ref, out_ref):
    o = jnp.maximum(y2_ref[...] * a_ref[...] + b_ref[...], 0.0)
    out_ref[...] = o.T


def _norm(y2, a2, b2):
    return pl.pallas_call(
        _norm_body,
        grid=(GRID,),
        in_specs=[
            pl.BlockSpec((BLK, 256), lambda i: (i, 0)),
            pl.BlockSpec((1, 256), lambda i: (0, 0)),
            pl.BlockSpec((1, 256), lambda i: (0, 0)),
        ],
        out_specs=pl.BlockSpec((256, BLK), lambda i: (0, i)),
        out_shape=jax.ShapeDtypeStruct((256, N), jnp.float32),
    )(y2, a2, b2)


def _bn_coefs(s, q, gamma, beta):
    mean = s / N
    var = q / N - mean * mean
    a = gamma[None, :] / jnp.sqrt(var + 1e-5)
    b = beta[None, :] - mean * a
    return a, b


def kernel(xyz_fine, xyz_coarse, feats_fine, feats_coarse,
           W1, gamma1, beta1, W2, gamma2, beta2):
    q = xyz_fine[0].T
    s = xyz_coarse[0]
    idx = _knn(q, s)

    table = jnp.pad(
        jnp.concatenate([feats_coarse[0].T, s.T], axis=1),
        ((0, 0), (0, FW - CC - 3)))
    idx_flat = idx.reshape(_ROWS)
    g_rows = _sc_gather(table, idx_flat)
    g = g_rows.reshape(N, K, FW)

    w1a_t = W1[:, :CC].T
    w1b_t = W1[:, CC:].T
    y1, s1, q1 = _mlp1(g, q, feats_fine[0], w1a_t, w1b_t)
    a1, b1 = _bn_coefs(s1, q1, gamma1, beta1)

    y2, s2, q2 = _mlp2(y1, a1, b1, W2.T)
    a2, b2 = _bn_coefs(s2, q2, gamma2, beta2)

    out = _norm(y2, a2, b2)
    return out[None]


# speedup vs baseline: 1.2624x; 1.2624x over previous
import functools

import jax
import jax.numpy as jnp
from jax import lax
from jax.experimental import pallas as pl
from jax.experimental.pallas import tpu as pltpu
from jax.experimental.pallas import tpu_sc as plsc

N = 16384
M = 4096
CF = 128
CC = 256
K = 3
FW = 384
BLK = 256
GRID = N // BLK
CW = 128
NCH = M // CW


def _knn_body(q_ref, s_ref, idx_ref):
    q = q_ref[...]
    s = s_ref[...]
    qq = (q[:, 0:1] * q[:, 0:1] + q[:, 1:2] * q[:, 1:2]) + q[:, 2:3] * q[:, 2:3]
    ss = (s[0:1] * s[0:1] + s[1:2] * s[1:2]) + s[2:3] * s[2:3]
    qb = q.astype(jnp.bfloat16)
    sb = s.astype(jnp.bfloat16)
    qs = jnp.dot(qb, sb, preferred_element_type=jnp.float32)
    inf = jnp.float32(float("inf"))
    big = jnp.int32(2**30)
    v1 = jnp.full((BLK, CW), inf, jnp.float32)
    v2 = v1
    v3 = v1
    i1 = jnp.full((BLK, CW), big, jnp.int32)
    i2 = i1
    i3 = i1
    lane = lax.broadcasted_iota(jnp.int32, (BLK, CW), 1)
    for c in range(NCH):
        sl = slice(c * CW, (c + 1) * CW)
        v = (qq + ss[:, sl]) - 2.0 * qs[:, sl]
        iv = lane + (c * CW)
        lt1 = v < v1
        lt2 = v < v2
        lt3 = v < v3
        v3n = jnp.where(lt3, jnp.where(lt2, v2, v), v3)
        i3n = jnp.where(lt3, jnp.where(lt2, i2, iv), i3)
        v2n = jnp.where(lt2, jnp.where(lt1, v1, v), v2)
        i2n = jnp.where(lt2, jnp.where(lt1, i1, iv), i2)
        v1 = jnp.where(lt1, v, v1)
        i1 = jnp.where(lt1, iv, i1)
        v2, v3, i2, i3 = v2n, v3n, i2n, i3n
    V = jnp.concatenate([v1, v2, v3], axis=1)
    I = jnp.concatenate([i1, i2, i3], axis=1)
    idxs = []
    for _ in range(K):
        m = jnp.min(V, axis=1, keepdims=True)
        a = jnp.min(jnp.where(V == m, I, big), axis=1, keepdims=True)
        idxs.append(a)
        V = jnp.where((V == m) & (I == a), inf, V)
    idx_ref[...] = jnp.concatenate(idxs, axis=1)


def _knn(q, s):
    return pl.pallas_call(
        _knn_body,
        grid=(GRID,),
        in_specs=[
            pl.BlockSpec((BLK, 3), lambda i: (i, 0)),
            pl.BlockSpec((3, M), lambda i: (0, 0)),
        ],
        out_specs=pl.BlockSpec((BLK, K), lambda i: (i, 0)),
        out_shape=jax.ShapeDtypeStruct((N, K), jnp.int32),
    )(q, s)


_NC = 2
_NS = 16
_NW = _NC * _NS
_ROWS = K * N
_RPW = _ROWS // _NW
_CHUNK = 128
_NCHUNK = _RPW // _CHUNK


def _sc_gather(table, idx_flat):
    mesh = plsc.VectorSubcoreMesh(core_axis_name="c", subcore_axis_name="s")

    @functools.partial(
        pl.kernel,
        mesh=mesh,
        out_type=jax.ShapeDtypeStruct((_ROWS, FW), jnp.float32),
        scratch_types=[
            pltpu.VMEM((_RPW,), jnp.int32),
            pltpu.VMEM((_CHUNK, FW), jnp.float32),
            pltpu.VMEM((_CHUNK, FW), jnp.float32),
            pltpu.SemaphoreType.DMA,
            pltpu.SemaphoreType.DMA,
        ],
    )
    def k(table_hbm, idx_hbm, out_hbm, idx_v, b0, b1, s0, s1):
        wid = lax.axis_index("s") * _NC + lax.axis_index("c")
        base = wid * _RPW
        pltpu.sync_copy(idx_hbm.at[pl.ds(base, _RPW)], idx_v)
        bufs = (b0, b1)
        sems = (s0, s1)

        def issue(t, b):
            isl = idx_v.at[pl.ds(t * _CHUNK, _CHUNK)]
            return pltpu.async_copy(table_hbm.at[isl], bufs[b], sems[b])

        cps = [None, None]
        cps[0] = issue(0, 0)
        for t in range(_NCHUNK):
            b = t % 2
            nb = (t + 1) % 2
            if t + 1 < _NCHUNK:
                cps[nb] = issue(t + 1, nb)
            cps[b].wait()
            pltpu.sync_copy(bufs[b], out_hbm.at[pl.ds(base + t * _CHUNK, _CHUNK)])

    return k(table, idx_flat)


def _mlp1_body(g_ref, qp_ref, ff_ref, w1a_ref, w1b_ref,
               y1_ref, s_ref, q_ref):
    i = pl.program_id(0)
    qp = qp_ref[...]
    ws = []
    for j in range(K):
        t0 = qp[:, 0:1] - g_ref[j, :, CC : CC + 1]
        t1 = qp[:, 1:2] - g_ref[j, :, CC + 1 : CC + 2]
        t2 = qp[:, 2:3] - g_ref[j, :, CC + 2 : CC + 3]
        d2 = (t0 * t0 + t1 * t1) + t2 * t2
        d = jnp.maximum(jnp.sqrt(d2), 1e-8)
        ws.append(1.0 / d)
    tot = (ws[0] + ws[1]) + ws[2]
    wn = [wj / tot for wj in ws]
    interp = (wn[0] * g_ref[0, :, :CC] + wn[1] * g_ref[1, :, :CC]) \
        + wn[2] * g_ref[2, :, :CC]
    y = jnp.dot(interp, w1a_ref[...], preferred_element_type=jnp.float32)
    y = y + lax.dot_general(
        ff_ref[...], w1b_ref[...],
        dimension_numbers=(((0,), (0,)), ((), ())),
        preferred_element_type=jnp.float32)
    y1_ref[...] = y

    @pl.when(i == 0)
    def _init():
        s_ref[...] = jnp.zeros_like(s_ref)
        q_ref[...] = jnp.zeros_like(q_ref)

    s_ref[...] += jnp.sum(y, axis=0, keepdims=True)
    q_ref[...] += jnp.sum(y * y, axis=0, keepdims=True)


def _mlp1(g, q, ff, w1a_t, w1b_t):
    return pl.pallas_call(
        _mlp1_body,
        grid=(GRID,),
        in_specs=[
            pl.BlockSpec((K, BLK, FW), lambda i: (0, i, 0)),
            pl.BlockSpec((BLK, 3), lambda i: (i, 0)),
            pl.BlockSpec((CF, BLK), lambda i: (0, i)),
            pl.BlockSpec((CC, 256), lambda i: (0, 0)),
            pl.BlockSpec((CF, 256), lambda i: (0, 0)),
        ],
        out_specs=[
            pl.BlockSpec((BLK, 256), lambda i: (i, 0)),
            pl.BlockSpec((1, 256), lambda i: (0, 0)),
            pl.BlockSpec((1, 256), lambda i: (0, 0)),
        ],
        out_shape=[
            jax.ShapeDtypeStruct((N, 256), jnp.float32),
            jax.ShapeDtypeStruct((1, 256), jnp.float32),
            jax.ShapeDtypeStruct((1, 256), jnp.float32),
        ],
    )(g, q, ff, w1a_t, w1b_t)


def _mlp2_body(y1_ref, a_ref, b_ref, w2_ref, y2_ref, s_ref, q_ref):
    i = pl.program_id(0)
    h = jnp.maximum(y1_ref[...] * a_ref[...] + b_ref[...], 0.0)
    y = jnp.dot(h, w2_ref[...], preferred_element_type=jnp.float32)
    y2_ref[...] = y

    @pl.when(i == 0)
    def _init():
        s_ref[...] = jnp.zeros_like(s_ref)
        q_ref[...] = jnp.zeros_like(q_ref)

    s_ref[...] += jnp.sum(y, axis=0, keepdims=True)
    q_ref[...] += jnp.sum(y * y, axis=0, keepdims=True)


def _mlp2(y1, a1, b1, w2_t):
    return pl.pallas_call(
        _mlp2_body,
        grid=(GRID,),
        in_specs=[
            pl.BlockSpec((BLK, 256), lambda i: (i, 0)),
            pl.BlockSpec((1, 256), lambda i: (0, 0)),
            pl.BlockSpec((1, 256), lambda i: (0, 0)),
            pl.BlockSpec((256, 256), lambda i: (0, 0)),
        ],
        out_specs=[
            pl.BlockSpec((BLK, 256), lambda i: (i, 0)),
            pl.BlockSpec((1, 256), lambda i: (0, 0)),
            pl.BlockSpec((1, 256), lambda i: (0, 0)),
        ],
        out_shape=[
            jax.ShapeDtypeStruct((N, 256), jnp.float32),
            jax.ShapeDtypeStruct((1, 256), jnp.float32),
            jax.ShapeDtypeStruct((1, 256), jnp.float32),
        ],
    )(y1, a1, b1, w2_t)


def _norm_body(y2_ref, a_ref, b_ref, out_ref):
    o = jnp.maximum(y2_ref[...] * a_ref[...] + b_ref[...], 0.0)
    out_ref[...] = o.T


def _norm(y2, a2, b2):
    return pl.pallas_call(
        _norm_body,
        grid=(GRID,),
        in_specs=[
            pl.BlockSpec((BLK, 256), lambda i: (i, 0)),
            pl.BlockSpec((1, 256), lambda i: (0, 0)),
            pl.BlockSpec((1, 256), lambda i: (0, 0)),
        ],
        out_specs=pl.BlockSpec((256, BLK), lambda i: (0, i)),
        out_shape=jax.ShapeDtypeStruct((256, N), jnp.float32),
    )(y2, a2, b2)


def _bn_coefs(s, q, gamma, beta):
    mean = s / N
    var = q / N - mean * mean
    a = gamma[None, :] / jnp.sqrt(var + 1e-5)
    b = beta[None, :] - mean * a
    return a, b


def kernel(xyz_fine, xyz_coarse, feats_fine, feats_coarse,
           W1, gamma1, beta1, W2, gamma2, beta2):
    q = xyz_fine[0].T
    s = xyz_coarse[0]
    idx = _knn(q, s)

    table = jnp.pad(
        jnp.concatenate([feats_coarse[0].T, s.T], axis=1),
        ((0, 0), (0, FW - CC - 3)))
    idx_flat = idx.T.reshape(_ROWS)
    g_rows = _sc_gather(table, idx_flat)
    g = g_rows.reshape(K, N, FW)

    w1a_t = W1[:, :CC].T
    w1b_t = W1[:, CC:].T
    y1, s1, q1 = _mlp1(g, q, feats_fine[0], w1a_t, w1b_t)
    a1, b1 = _bn_coefs(s1, q1, gamma1, beta1)

    y2, s2, q2 = _mlp2(y1, a1, b1, W2.T)
    a2, b2 = _bn_coefs(s2, q2, gamma2, beta2)

    out = _norm(y2, a2, b2)
    return out[None]


# speedup vs baseline: 1.2666x; 1.0033x over previous
import functools

import jax
import jax.numpy as jnp
from jax import lax
from jax.experimental import pallas as pl
from jax.experimental.pallas import tpu as pltpu
from jax.experimental.pallas import tpu_sc as plsc

N = 16384
M = 4096
CF = 128
CC = 256
K = 3
FW = 384
BLK = 256
GRID = N // BLK
CW = 128
NCH = M // CW


def _knn_body(q_ref, s_ref, idx_ref):
    q = q_ref[...]
    s = s_ref[...]
    qq = (q[:, 0:1] * q[:, 0:1] + q[:, 1:2] * q[:, 1:2]) + q[:, 2:3] * q[:, 2:3]
    ss = (s[0:1] * s[0:1] + s[1:2] * s[1:2]) + s[2:3] * s[2:3]
    qb = q.astype(jnp.bfloat16)
    sb = s.astype(jnp.bfloat16)
    qs = jnp.dot(qb, sb, preferred_element_type=jnp.float32)
    inf = jnp.float32(float("inf"))
    big = jnp.int32(2**30)
    v1 = jnp.full((BLK, CW), inf, jnp.float32)
    v2 = v1
    v3 = v1
    i1 = jnp.full((BLK, CW), big, jnp.int32)
    i2 = i1
    i3 = i1
    lane = lax.broadcasted_iota(jnp.int32, (BLK, CW), 1)
    for c in range(NCH):
        sl = slice(c * CW, (c + 1) * CW)
        v = (qq + ss[:, sl]) - 2.0 * qs[:, sl]
        iv = lane + (c * CW)
        lt1 = v < v1
        lt2 = v < v2
        lt3 = v < v3
        v3n = jnp.where(lt3, jnp.where(lt2, v2, v), v3)
        i3n = jnp.where(lt3, jnp.where(lt2, i2, iv), i3)
        v2n = jnp.where(lt2, jnp.where(lt1, v1, v), v2)
        i2n = jnp.where(lt2, jnp.where(lt1, i1, iv), i2)
        v1 = jnp.where(lt1, v, v1)
        i1 = jnp.where(lt1, iv, i1)
        v2, v3, i2, i3 = v2n, v3n, i2n, i3n
    V = jnp.concatenate([v1, v2, v3], axis=1)
    I = jnp.concatenate([i1, i2, i3], axis=1)
    idxs = []
    for _ in range(K):
        m = jnp.min(V, axis=1, keepdims=True)
        a = jnp.min(jnp.where(V == m, I, big), axis=1, keepdims=True)
        idxs.append(a)
        V = jnp.where((V == m) & (I == a), inf, V)
    idx_ref[...] = jnp.concatenate(idxs, axis=1)


def _knn(q, s):
    return pl.pallas_call(
        _knn_body,
        grid=(GRID,),
        in_specs=[
            pl.BlockSpec((BLK, 3), lambda i: (i, 0)),
            pl.BlockSpec((3, M), lambda i: (0, 0)),
        ],
        out_specs=pl.BlockSpec((BLK, K), lambda i: (i, 0)),
        out_shape=jax.ShapeDtypeStruct((N, K), jnp.int32),
    )(q, s)


_NC = 2
_NS = 16
_NW = _NC * _NS
_ROWS = K * N
_RPW = _ROWS // _NW
_CHUNK = 128
_NCHUNK = _RPW // _CHUNK


def _sc_gather(table, idx_flat):
    mesh = plsc.VectorSubcoreMesh(core_axis_name="c", subcore_axis_name="s")

    @functools.partial(
        pl.kernel,
        mesh=mesh,
        out_type=jax.ShapeDtypeStruct((_ROWS, FW), jnp.float32),
        scratch_types=[
            pltpu.VMEM((_RPW,), jnp.int32),
            pltpu.VMEM((_CHUNK, FW), jnp.float32),
            pltpu.VMEM((_CHUNK, FW), jnp.float32),
            pltpu.SemaphoreType.DMA,
            pltpu.SemaphoreType.DMA,
        ],
    )
    def k(table_hbm, idx_hbm, out_hbm, idx_v, b0, b1, s0, s1):
        wid = lax.axis_index("s") * _NC + lax.axis_index("c")
        base = wid * _RPW
        pltpu.sync_copy(idx_hbm.at[pl.ds(base, _RPW)], idx_v)
        bufs = (b0, b1)
        sems = (s0, s1)

        def issue(t, b):
            isl = idx_v.at[pl.ds(t * _CHUNK, _CHUNK)]
            return pltpu.async_copy(table_hbm.at[isl], bufs[b], sems[b])

        cps = [None, None]
        cps[0] = issue(0, 0)
        for t in range(_NCHUNK):
            b = t % 2
            nb = (t + 1) % 2
            if t + 1 < _NCHUNK:
                cps[nb] = issue(t + 1, nb)
            cps[b].wait()
            pltpu.sync_copy(bufs[b], out_hbm.at[pl.ds(base + t * _CHUNK, _CHUNK)])

    return k(table, idx_flat)


def _mlp1_body(g_ref, qp_ref, ff_ref, w1a_ref, w1b_ref,
               y1_ref, s_ref, q_ref):
    i = pl.program_id(0)
    qp = qp_ref[...]
    ws = []
    for j in range(K):
        t0 = qp[:, 0:1] - g_ref[j, :, CC : CC + 1]
        t1 = qp[:, 1:2] - g_ref[j, :, CC + 1 : CC + 2]
        t2 = qp[:, 2:3] - g_ref[j, :, CC + 2 : CC + 3]
        d2 = (t0 * t0 + t1 * t1) + t2 * t2
        d = jnp.maximum(jnp.sqrt(d2), 1e-8)
        ws.append(1.0 / d)
    tot = (ws[0] + ws[1]) + ws[2]
    wn = [wj / tot for wj in ws]
    interp = (wn[0] * g_ref[0, :, :CC] + wn[1] * g_ref[1, :, :CC]) \
        + wn[2] * g_ref[2, :, :CC]
    y = jnp.dot(interp, w1a_ref[...], preferred_element_type=jnp.float32)
    y = y + lax.dot_general(
        ff_ref[...], w1b_ref[...],
        dimension_numbers=(((0,), (0,)), ((), ())),
        preferred_element_type=jnp.float32)
    y1_ref[...] = y

    @pl.when(i == 0)
    def _init():
        s_ref[...] = jnp.zeros_like(s_ref)
        q_ref[...] = jnp.zeros_like(q_ref)

    s_ref[...] += jnp.sum(y, axis=0, keepdims=True)
    q_ref[...] += jnp.sum(y * y, axis=0, keepdims=True)


def _mlp1(g, q, ff, w1a_t, w1b_t):
    return pl.pallas_call(
        _mlp1_body,
        grid=(GRID,),
        in_specs=[
            pl.BlockSpec((K, BLK, FW), lambda i: (0, i, 0)),
            pl.BlockSpec((BLK, 3), lambda i: (i, 0)),
            pl.BlockSpec((CF, BLK), lambda i: (0, i)),
            pl.BlockSpec((CC, 256), lambda i: (0, 0)),
            pl.BlockSpec((CF, 256), lambda i: (0, 0)),
        ],
        out_specs=[
            pl.BlockSpec((BLK, 256), lambda i: (i, 0)),
            pl.BlockSpec((1, 256), lambda i: (0, 0)),
            pl.BlockSpec((1, 256), lambda i: (0, 0)),
        ],
        out_shape=[
            jax.ShapeDtypeStruct((N, 256), jnp.float32),
            jax.ShapeDtypeStruct((1, 256), jnp.float32),
            jax.ShapeDtypeStruct((1, 256), jnp.float32),
        ],
    )(g, q, ff, w1a_t, w1b_t)


def _bn_ab(s, q, gamma, beta):
    mean = s * (1.0 / N)
    var = q * (1.0 / N) - mean * mean
    a = gamma / jnp.sqrt(var + 1e-5)
    return a, beta - mean * a


def _mlp2_body(y1_ref, s1_ref, q1_ref, g1_ref, b1_ref, w2_ref,
               y2_ref, s_ref, q_ref):
    i = pl.program_id(0)
    a_, b_ = _bn_ab(s1_ref[...], q1_ref[...], g1_ref[...], b1_ref[...])
    h = jnp.maximum(y1_ref[...] * a_ + b_, 0.0)
    y = jnp.dot(h, w2_ref[...], preferred_element_type=jnp.float32)
    y2_ref[...] = y

    @pl.when(i == 0)
    def _init():
        s_ref[...] = jnp.zeros_like(s_ref)
        q_ref[...] = jnp.zeros_like(q_ref)

    s_ref[...] += jnp.sum(y, axis=0, keepdims=True)
    q_ref[...] += jnp.sum(y * y, axis=0, keepdims=True)


def _mlp2(y1, s1, q1, g1, b1, w2_t):
    return pl.pallas_call(
        _mlp2_body,
        grid=(GRID,),
        in_specs=[
            pl.BlockSpec((BLK, 256), lambda i: (i, 0)),
            pl.BlockSpec((1, 256), lambda i: (0, 0)),
            pl.BlockSpec((1, 256), lambda i: (0, 0)),
            pl.BlockSpec((1, 256), lambda i: (0, 0)),
            pl.BlockSpec((1, 256), lambda i: (0, 0)),
            pl.BlockSpec((256, 256), lambda i: (0, 0)),
        ],
        out_specs=[
            pl.BlockSpec((BLK, 256), lambda i: (i, 0)),
            pl.BlockSpec((1, 256), lambda i: (0, 0)),
            pl.BlockSpec((1, 256), lambda i: (0, 0)),
        ],
        out_shape=[
            jax.ShapeDtypeStruct((N, 256), jnp.float32),
            jax.ShapeDtypeStruct((1, 256), jnp.float32),
            jax.ShapeDtypeStruct((1, 256), jnp.float32),
        ],
    )(y1, s1, q1, g1, b1, w2_t)


def _norm_body(y2_ref, s2_ref, q2_ref, g2_ref, b2_ref, out_ref):
    a_, b_ = _bn_ab(s2_ref[...], q2_ref[...], g2_ref[...], b2_ref[...])
    o = jnp.maximum(y2_ref[...] * a_ + b_, 0.0)
    out_ref[...] = o.T


def _norm(y2, s2, q2, g2, b2):
    return pl.pallas_call(
        _norm_body,
        grid=(GRID,),
        in_specs=[
            pl.BlockSpec((BLK, 256), lambda i: (i, 0)),
            pl.BlockSpec((1, 256), lambda i: (0, 0)),
            pl.BlockSpec((1, 256), lambda i: (0, 0)),
            pl.BlockSpec((1, 256), lambda i: (0, 0)),
            pl.BlockSpec((1, 256), lambda i: (0, 0)),
        ],
        out_specs=pl.BlockSpec((256, BLK), lambda i: (0, i)),
        out_shape=jax.ShapeDtypeStruct((256, N), jnp.float32),
    )(y2, s2, q2, g2, b2)


def kernel(xyz_fine, xyz_coarse, feats_fine, feats_coarse,
           W1, gamma1, beta1, W2, gamma2, beta2):
    q = xyz_fine[0].T
    s = xyz_coarse[0]
    idx = _knn(q, s)

    table = jnp.pad(
        jnp.concatenate([feats_coarse[0].T, s.T], axis=1),
        ((0, 0), (0, FW - CC - 3)))
    idx_flat = idx.T.reshape(_ROWS)
    g_rows = _sc_gather(table, idx_flat)
    g = g_rows.reshape(K, N, FW)

    w1a_t = W1[:, :CC].T
    w1b_t = W1[:, CC:].T
    y1, s1, q1 = _mlp1(g, q, feats_fine[0], w1a_t, w1b_t)
    y2, s2, q2 = _mlp2(y1, s1, q1, gamma1[None], beta1[None], W2.T)
    out = _norm(y2, s2, q2, gamma2[None], beta2[None])
    return out[None]
